# baseline (speedup 1.0000x reference)
import functools
import math

import jax
import jax.numpy as jnp
from jax import lax
from jax.experimental import pallas as pl
from jax.experimental.pallas import tpu as pltpu
from jax.experimental.pallas import tpu_sc as plsc

D = 128
NC, NS = 2, 16
NW = NC * NS
_HALF_PI = 0.5 * math.pi
_LOG1E4 = math.log(10000.0)
_BE = 3200
_BN = 2000


def _chunk_rows(epw):
    for c in range(128, 0, -8):
        if epw % c == 0:
            return c
    raise ValueError(epw)

_F32 = jnp.float32


def _ln(x, w, b, eps=1e-5):
    mu = jnp.mean(x, axis=-1, keepdims=True)
    var = jnp.mean((x - mu) * (x - mu), axis=-1, keepdims=True)
    return (x - mu) * lax.rsqrt(var + eps) * w + b


def _sincos_rows(a, b):
    rows = a.shape[0]
    j = lax.broadcasted_iota(jnp.int32, (rows, D), 1)
    f = jnp.exp((j % 32).astype(_F32) * (-_LOG1E4 / 32.0))
    val = jnp.where(j < 64, a, b)
    phase = jnp.where((j // 32) % 2 == 1, _HALF_PI, 0.0).astype(_F32)
    return jnp.sin(val * f + phase)


def _dot(a, b):
    return jnp.dot(a, b, preferred_element_type=_F32)


def _wspec(*shape):
    return pl.BlockSpec(shape, lambda i: tuple(0 for _ in shape))


def _rowspec(rows, cols):
    return pl.BlockSpec((rows, cols), lambda i: (i, 0))


def _tmlp_body(temb_r, tpw0, tpb0, tpw1, tpb1, tmw0, tmb0, tmw1, tmb1, out_r):
    x = temb_r[...]
    x = _dot(x, tpw0[...]) + tpb0[...]
    x = x * jax.nn.sigmoid(x)
    x = _dot(x, tpw1[...]) + tpb1[...]
    n_layers = out_r.shape[0]
    for l in range(n_layers):
        y = jnp.maximum(_dot(x, tmw0[l]) + tmb0[l], 0.0)
        out_r[l] = _dot(y, tmw1[l]) + tmb1[l]


def _embed_e_body(d_r, x_r, out_r):
    out_r[...] = _sincos_rows(d_r[...], x_r[...])


def _node0_body(coords_r, q, u, h_o, a_o, uh_o):
    h = _sincos_rows(coords_r[:, 0:1], coords_r[:, 1:2])
    h_o[...] = h
    a_o[...] = _dot(h, q[...])
    uh_o[...] = _dot(h, u[...])


def _node_body(h_r, uh_r, g0, g1, g2, g3, nw, nb, q, u, h_o, a_o, uh_o):
    x = uh_r[...] + (g0[...] + g1[...]) + (g2[...] + g3[...])
    h = h_r[...] + jnp.maximum(_ln(x, nw[...], nb[...]), 0.0)
    h_o[...] = h
    a_o[...] = _dot(h, q[...])
    uh_o[...] = _dot(h, u[...])


def _node_last_body(h_r, uh_r, g0, g1, g2, g3, nw, nb, q, h_o, a_o):
    x = uh_r[...] + (g0[...] + g1[...]) + (g2[...] + g3[...])
    h = h_r[...] + jnp.maximum(_ln(x, nw[...], nb[...]), 0.0)
    h_o[...] = h
    a_o[...] = _dot(h, q[...])


def _edge_body(e_r, ga_r, gb_r, p, rv, w1, c1, w2, c2, enw, enb, trow, eo, mo):
    e = e_r[...]
    hv = _dot(gb_r[...], rv[...])
    ehat = _dot(e, p[...]) + ga_r[...] + hv[:, :D]
    en = _ln(ehat, enw[...], enb[...])
    m1 = jnp.maximum(_dot(en, w1[...]) + c1[...], 0.0)
    emlp = _dot(m1, w2[...]) + c2[...]
    eo[...] = e + emlp + trow[...]
    mo[...] = jax.nn.sigmoid(ehat) * hv[:, D:]


def _edge_last_body(e_r, ga_r, gb_r, p, r, w1, c1, w2, c2, enw, enb, trow, eo):
    e = e_r[...]
    ehat = _dot(e, p[...]) + ga_r[...] + _dot(gb_r[...], r[...])
    en = _ln(ehat, enw[...], enb[...])
    m1 = jnp.maximum(_dot(en, w1[...]) + c1[...], 0.0)
    emlp = _dot(m1, w2[...]) + c2[...]
    eo[...] = e + emlp + trow[...]


def _head_body(e_r, w1, c1, w2, c2, out_r):
    x = jnp.maximum(_dot(e_r[...], w1[...]) + c1[...], 0.0)
    out_r[...] = _dot(x, w2[...]) + c2[...]


@functools.cache
def _make_gather(n_edges):
    epw = n_edges // NW
    ch_rows = _chunk_rows(epw)
    nch = epw // ch_rows
    mesh = plsc.VectorSubcoreMesh(core_axis_name="c", subcore_axis_name="s")

    @functools.partial(
        pl.kernel,
        out_type=(jax.ShapeDtypeStruct((n_edges, D), _F32),
                  jax.ShapeDtypeStruct((n_edges, D), _F32)),
        mesh=mesh,
        scratch_types=[
            pltpu.VMEM((ch_rows,), jnp.int32),
            pltpu.VMEM((ch_rows,), jnp.int32),
            pltpu.VMEM((ch_rows, D), _F32),
            pltpu.VMEM((ch_rows, D), _F32),
            pltpu.SemaphoreType.DMA,
            pltpu.SemaphoreType.DMA,
        ],
    )
    def gather(a_hbm, b_hbm, src_hbm, dst_hbm, ga_hbm, gb_hbm,
               idx_s, idx_d, buf_a, buf_b, sem_a, sem_b):
        wid = lax.axis_index("s") * NC + lax.axis_index("c")
        base = wid * epw

        @pl.loop(0, nch)
        def _(ch):
            off = base + ch * ch_rows
            pltpu.sync_copy(src_hbm.at[pl.ds(off, ch_rows)], idx_s)
            pltpu.sync_copy(dst_hbm.at[pl.ds(off, ch_rows)], idx_d)
            ca = pltpu.async_copy(a_hbm.at[idx_s], buf_a, sem_a)
            cb = pltpu.async_copy(b_hbm.at[idx_d], buf_b, sem_b)
            ca.wait()
            cb.wait()
            pltpu.sync_copy(buf_a, ga_hbm.at[pl.ds(off, ch_rows)])
            pltpu.sync_copy(buf_b, gb_hbm.at[pl.ds(off, ch_rows)])

    return gather


@functools.cache
def _make_scatter(n_edges, n_nodes):
    epw = n_edges // NW
    ch_rows = _chunk_rows(epw)
    nch = epw // ch_rows
    rps = n_nodes // NS
    mesh = plsc.VectorSubcoreMesh(core_axis_name="c", subcore_axis_name="s")

    @functools.partial(
        pl.kernel,
        out_type=jax.ShapeDtypeStruct((NC, n_nodes, D), _F32),
        mesh=mesh,
        scratch_types=[
            pltpu.VMEM((ch_rows,), jnp.int32),
            pltpu.VMEM((ch_rows, D), _F32),
            pltpu.VMEM_SHARED((n_nodes, D), _F32),
            pltpu.SemaphoreType.DMA,
        ],
    )
    def scatter(msg_hbm, src_hbm, zeros_hbm, out_hbm, idx_v, buf, acc, sem):
        c = lax.axis_index("c")
        s = lax.axis_index("s")

        @pl.when(s == 0)
        def _():
            pltpu.sync_copy(zeros_hbm, acc)

        plsc.subcore_barrier()
        base = (s * NC + c) * epw

        @pl.loop(0, nch)
        def _(ch):
            off = base + ch * ch_rows
            pltpu.sync_copy(src_hbm.at[pl.ds(off, ch_rows)], idx_v)
            pltpu.async_copy(msg_hbm.at[pl.ds(off, ch_rows)], buf, sem).wait()
            pltpu.sync_copy(buf, acc.at[idx_v], add=True)

        plsc.subcore_barrier()
        pltpu.sync_copy(acc.at[pl.ds(s * rps, rps)],
                        out_hbm.at[c, pl.ds(s * rps, rps)])

    return scatter


def _gather_tables(a, b, src, dst):
    return _make_gather(src.shape[0])(a, b, src, dst)


def _scatter_msg(msg, src, zeros):
    return _make_scatter(src.shape[0], zeros.shape[0])(msg, src, zeros)


def _w(p):
    return p["W"]


def _b2d(p):
    return p["b"].reshape(1, D)


def kernel(node_coords, edge_index, edge_distances, x_t, t, params):
    n_nodes = node_coords.shape[0]
    n_edges = edge_index.shape[1]
    layers = params["layers"]
    n_layers = len(layers)
    e_half = n_edges // 2
    srcs = (edge_index[0, :e_half], edge_index[0, e_half:])
    dsts = (edge_index[1, :e_half], edge_index[1, e_half:])
    eg = e_half // _BE
    ng = n_nodes // _BN

    tf = t.astype(_F32)
    f64 = jnp.exp(-_LOG1E4 * jnp.arange(D // 2, dtype=_F32) / (D // 2))
    targ = tf[:, None] * f64
    temb8 = jnp.broadcast_to(
        jnp.concatenate([jnp.sin(targ), jnp.cos(targ)], axis=-1), (8, D))
    tmw0 = jnp.stack([_w(lp["time_mlp"][0]) for lp in layers])
    tmb0 = jnp.stack([_b2d(lp["time_mlp"][0]) for lp in layers])
    tmw1 = jnp.stack([_w(lp["time_mlp"][1]) for lp in layers])
    tmb1 = jnp.stack([_b2d(lp["time_mlp"][1]) for lp in layers])
    trows = pl.pallas_call(
        _tmlp_body,
        grid=(1,),
        in_specs=[_wspec(8, D), _wspec(D, D), _wspec(1, D), _wspec(D, D),
                  _wspec(1, D), _wspec(n_layers, D, D), _wspec(n_layers, 1, D),
                  _wspec(n_layers, D, D), _wspec(n_layers, 1, D)],
        out_specs=_wspec(n_layers, 8, D),
        out_shape=jax.ShapeDtypeStruct((n_layers, 8, D), _F32),
    )(temb8, _w(params["time_proj"][0]), _b2d(params["time_proj"][0]),
      _w(params["time_proj"][1]), _b2d(params["time_proj"][1]),
      tmw0, tmb0, tmw1, tmb1)

    dist2 = edge_distances.reshape(n_edges, 1)
    xt2 = x_t.reshape(n_edges, 1)
    es = [
        pl.pallas_call(
            _embed_e_body,
            grid=(eg,),
            in_specs=[_rowspec(_BE, 1), _rowspec(_BE, 1)],
            out_specs=_rowspec(_BE, D),
            out_shape=jax.ShapeDtypeStruct((e_half, D), _F32),
        )(dist2[k * e_half:(k + 1) * e_half], xt2[k * e_half:(k + 1) * e_half])
        for k in range(2)
    ]

    n_pad = ((n_nodes + NS * 8 - 1) // (NS * 8)) * (NS * 8)
    zeros_nd = jnp.zeros((n_pad, D), _F32)
    wspec_dd = _wspec(D, D)
    wspec_1d = _wspec(1, D)

    h = uh = None
    aggs = None
    for l in range(n_layers):
        lp = layers[l]
        last = l == n_layers - 1
        if l == 0:
            h, a_tab, uh = pl.pallas_call(
                _node0_body,
                grid=(ng,),
                in_specs=[_rowspec(_BN, 2)] + [wspec_dd] * 2,
                out_specs=[_rowspec(_BN, D)] * 3,
                out_shape=[jax.ShapeDtypeStruct((n_nodes, D), _F32)] * 3,
            )(node_coords, _w(lp["Q"]), _w(lp["U"]))
        else:
            nn = layers[l - 1]["node_norm"]
            common = (h, uh, aggs[0][0], aggs[0][1], aggs[1][0], aggs[1][1],
                      nn["w"].reshape(1, D), nn["b"].reshape(1, D))
            if last:
                h, a_tab = pl.pallas_call(
                    _node_last_body,
                    grid=(ng,),
                    in_specs=[_rowspec(_BN, D)] * 6 + [wspec_1d] * 2
                             + [wspec_dd],
                    out_specs=[_rowspec(_BN, D)] * 2,
                    out_shape=[jax.ShapeDtypeStruct((n_nodes, D), _F32)] * 2,
                )(*common, _w(lp["Q"]))
            else:
                h, a_tab, uh = pl.pallas_call(
                    _node_body,
                    grid=(ng,),
                    in_specs=[_rowspec(_BN, D)] * 6 + [wspec_1d] * 2
                             + [wspec_dd] * 2,
                    out_specs=[_rowspec(_BN, D)] * 3,
                    out_shape=[jax.ShapeDtypeStruct((n_nodes, D), _F32)] * 3,
                )(*common, _w(lp["Q"]), _w(lp["U"]))

        trow = lax.slice_in_dim(trows[l], 0, 1, axis=0)
        rv = (_w(lp["R"]) if last
              else jnp.concatenate([_w(lp["R"]), _w(lp["V"])], axis=1))
        ew = (_w(lp["P"]), rv, _w(lp["edge_mlp"][0]), _b2d(lp["edge_mlp"][0]),
              _w(lp["edge_mlp"][1]), _b2d(lp["edge_mlp"][1]),
              lp["edge_norm"]["w"].reshape(1, D),
              lp["edge_norm"]["b"].reshape(1, D), trow)
        in_specs = [_rowspec(_BE, D), _rowspec(_BE, D), _rowspec(_BE, D),
                    wspec_dd, _wspec(D, D if last else 2 * D),
                    wspec_dd, wspec_1d, wspec_dd, wspec_1d,
                    wspec_1d, wspec_1d, wspec_1d]
        gs = [_gather_tables(a_tab, h, srcs[k], dsts[k]) for k in range(2)]
        aggs = []
        for k in range(2):
            ga, gb = gs[k]
            if last:
                es[k] = pl.pallas_call(
                    _edge_last_body,
                    grid=(eg,),
                    in_specs=in_specs,
                    out_specs=_rowspec(_BE, D),
                    out_shape=jax.ShapeDtypeStruct((e_half, D), _F32),
                )(es[k], ga, gb, *ew)
            else:
                es[k], msg = pl.pallas_call(
                    _edge_body,
                    grid=(eg,),
                    in_specs=in_specs,
                    out_specs=[_rowspec(_BE, D), _rowspec(_BE, D)],
                    out_shape=[jax.ShapeDtypeStruct((e_half, D), _F32),
                               jax.ShapeDtypeStruct((e_half, D), _F32)],
                )(es[k], ga, gb, *ew)
                aggs.append(_scatter_msg(msg, srcs[k], zeros_nd))

    hw2 = jnp.zeros((D, D), _F32).at[:, :2].set(_w(params["edge_head"][1]))
    hb2 = jnp.zeros((1, D), _F32).at[0, :2].set(params["edge_head"][1]["b"])
    outs = [
        pl.pallas_call(
            _head_body,
            grid=(eg,),
            in_specs=[_rowspec(_BE, D), wspec_dd, wspec_1d, wspec_dd,
                      wspec_1d],
            out_specs=_rowspec(_BE, D),
            out_shape=jax.ShapeDtypeStruct((e_half, D), _F32),
        )(es[k], _w(params["edge_head"][0]), _b2d(params["edge_head"][0]),
          hw2, hb2)
        for k in range(2)
    ]
    return jnp.concatenate([outs[0][:, :2], outs[1][:, :2]], axis=0)


# speedup vs baseline: 1.1845x; 1.1845x over previous
import functools
import math

import jax
import jax.numpy as jnp
from jax import lax
from jax.experimental import pallas as pl
from jax.experimental.pallas import tpu as pltpu
from jax.experimental.pallas import tpu_sc as plsc

D = 128
NC, NS = 2, 16
NW = NC * NS
_HALF_PI = 0.5 * math.pi
_LOG1E4 = math.log(10000.0)
_BE = 2560
_BN = 2000


def _chunk_rows(epw):
    for c in range(128, 0, -8):
        if epw % c == 0:
            return c
    raise ValueError(epw)

_F32 = jnp.float32


def _ln(x, w, b, eps=1e-5):
    mu = jnp.mean(x, axis=-1, keepdims=True)
    var = jnp.mean((x - mu) * (x - mu), axis=-1, keepdims=True)
    return (x - mu) * lax.rsqrt(var + eps) * w + b


def _sincos_rows(a, b):
    rows = a.shape[0]
    j = lax.broadcasted_iota(jnp.int32, (rows, D), 1)
    f = jnp.exp((j % 32).astype(_F32) * (-_LOG1E4 / 32.0))
    val = jnp.where(j < 64, a, b)
    phase = jnp.where((j // 32) % 2 == 1, _HALF_PI, 0.0).astype(_F32)
    return jnp.sin(val * f + phase)


def _dot(a, b):
    return jnp.dot(a, b, preferred_element_type=_F32)


def _wspec(*shape):
    return pl.BlockSpec(shape, lambda i: tuple(0 for _ in shape))


def _rowspec(rows, cols):
    return pl.BlockSpec((rows, cols), lambda i: (i, 0))


def _tmlp_body(temb_r, tpw0, tpb0, tpw1, tpb1, tmw0, tmb0, tmw1, tmb1, out_r):
    x = temb_r[...]
    x = _dot(x, tpw0[...]) + tpb0[...]
    x = x * jax.nn.sigmoid(x)
    x = _dot(x, tpw1[...]) + tpb1[...]
    n_layers = out_r.shape[0]
    for l in range(n_layers):
        y = jnp.maximum(_dot(x, tmw0[l]) + tmb0[l], 0.0)
        out_r[l] = _dot(y, tmw1[l]) + tmb1[l]


def _embed_e_body(d_r, x_r, out_r):
    out_r[...] = _sincos_rows(d_r[...], x_r[...])


def _node0_body(coords_r, q, u, h_o, a_o, uh_o):
    h = _sincos_rows(coords_r[:, 0:1], coords_r[:, 1:2])
    h_o[...] = h
    a_o[...] = _dot(h, q[...])
    uh_o[...] = _dot(h, u[...])


def _node_body(h_r, uh_r, g0, g1, nw, nb, q, u, h_o, a_o, uh_o):
    x = uh_r[...] + g0[...] + g1[...]
    h = h_r[...] + jnp.maximum(_ln(x, nw[...], nb[...]), 0.0)
    h_o[...] = h
    a_o[...] = _dot(h, q[...])
    uh_o[...] = _dot(h, u[...])


def _node_last_body(h_r, uh_r, g0, g1, nw, nb, q, h_o, a_o):
    x = uh_r[...] + g0[...] + g1[...]
    h = h_r[...] + jnp.maximum(_ln(x, nw[...], nb[...]), 0.0)
    h_o[...] = h
    a_o[...] = _dot(h, q[...])


def _edge_body(e_r, ga_r, gb_r, p, rv, w1, c1, w2, c2, enw, enb, trow, eo, mo):
    e = e_r[...]
    hv = _dot(gb_r[...], rv[...])
    ehat = _dot(e, p[...]) + ga_r[...] + hv[:, :D]
    en = _ln(ehat, enw[...], enb[...])
    m1 = jnp.maximum(_dot(en, w1[...]) + c1[...], 0.0)
    emlp = _dot(m1, w2[...]) + c2[...]
    eo[...] = e + emlp + trow[...]
    mo[...] = jax.nn.sigmoid(ehat) * hv[:, D:]


def _edge_last_body(e_r, ga_r, gb_r, p, r, w1, c1, w2, c2, enw, enb, trow, eo):
    e = e_r[...]
    ehat = _dot(e, p[...]) + ga_r[...] + _dot(gb_r[...], r[...])
    en = _ln(ehat, enw[...], enb[...])
    m1 = jnp.maximum(_dot(en, w1[...]) + c1[...], 0.0)
    emlp = _dot(m1, w2[...]) + c2[...]
    eo[...] = e + emlp + trow[...]


def _head_body(e_r, w1, c1, w2, c2, out_r):
    x = jnp.maximum(_dot(e_r[...], w1[...]) + c1[...], 0.0)
    out_r[...] = _dot(x, w2[...]) + c2[...]


@functools.cache
def _make_gather(n_edges):
    epw = n_edges // NW
    ch_rows = _chunk_rows(epw)
    nch = epw // ch_rows
    mesh = plsc.VectorSubcoreMesh(core_axis_name="c", subcore_axis_name="s")

    @functools.partial(
        pl.kernel,
        out_type=(jax.ShapeDtypeStruct((n_edges, D), _F32),
                  jax.ShapeDtypeStruct((n_edges, D), _F32)),
        mesh=mesh,
        scratch_types=[
            pltpu.VMEM((ch_rows,), jnp.int32),
            pltpu.VMEM((ch_rows,), jnp.int32),
            pltpu.VMEM((ch_rows, D), _F32),
            pltpu.VMEM((ch_rows, D), _F32),
            pltpu.SemaphoreType.DMA,
            pltpu.SemaphoreType.DMA,
        ],
    )
    def gather(a_hbm, b_hbm, src_hbm, dst_hbm, ga_hbm, gb_hbm,
               idx_s, idx_d, buf_a, buf_b, sem_a, sem_b):
        wid = lax.axis_index("s") * NC + lax.axis_index("c")
        base = wid * epw

        @pl.loop(0, nch)
        def _(ch):
            off = base + ch * ch_rows
            pltpu.sync_copy(src_hbm.at[pl.ds(off, ch_rows)], idx_s)
            pltpu.sync_copy(dst_hbm.at[pl.ds(off, ch_rows)], idx_d)
            ca = pltpu.async_copy(a_hbm.at[idx_s], buf_a, sem_a)
            cb = pltpu.async_copy(b_hbm.at[idx_d], buf_b, sem_b)
            ca.wait()
            cb.wait()
            pltpu.sync_copy(buf_a, ga_hbm.at[pl.ds(off, ch_rows)])
            pltpu.sync_copy(buf_b, gb_hbm.at[pl.ds(off, ch_rows)])

    return gather


@functools.cache
def _make_scatter(n_edges, n_nodes):
    epw = n_edges // NW
    ch_rows = _chunk_rows(epw)
    nch = epw // ch_rows
    rps = n_nodes // NS
    mesh = plsc.VectorSubcoreMesh(core_axis_name="c", subcore_axis_name="s")

    @functools.partial(
        pl.kernel,
        out_type=jax.ShapeDtypeStruct((NC, n_nodes, D), _F32),
        mesh=mesh,
        scratch_types=[
            pltpu.VMEM((ch_rows,), jnp.int32),
            pltpu.VMEM((ch_rows, D), _F32),
            pltpu.VMEM_SHARED((n_nodes, D), _F32),
            pltpu.SemaphoreType.DMA,
        ],
    )
    def scatter(msg_hbm, src_hbm, zeros_hbm, out_hbm, idx_v, buf, acc, sem):
        c = lax.axis_index("c")
        s = lax.axis_index("s")

        @pl.when(s == 0)
        def _():
            pltpu.sync_copy(zeros_hbm, acc)

        plsc.subcore_barrier()
        base = (s * NC + c) * epw

        @pl.loop(0, nch)
        def _(ch):
            off = base + ch * ch_rows
            pltpu.sync_copy(src_hbm.at[pl.ds(off, ch_rows)], idx_v)
            pltpu.async_copy(msg_hbm.at[pl.ds(off, ch_rows)], buf, sem).wait()
            pltpu.sync_copy(buf, acc.at[idx_v], add=True)

        plsc.subcore_barrier()
        pltpu.sync_copy(acc.at[pl.ds(s * rps, rps)],
                        out_hbm.at[c, pl.ds(s * rps, rps)])

    return scatter


def _gather_tables(a, b, src, dst):
    return _make_gather(src.shape[0])(a, b, src, dst)


def _scatter_msg(msg, src, zeros):
    return _make_scatter(src.shape[0], zeros.shape[0])(msg, src, zeros)


def _w(p):
    return p["W"]


def _b2d(p):
    return p["b"].reshape(1, D)


def kernel(node_coords, edge_index, edge_distances, x_t, t, params):
    n_nodes = node_coords.shape[0]
    n_edges = edge_index.shape[1]
    layers = params["layers"]
    n_layers = len(layers)
    src = edge_index[0]
    dst = edge_index[1]
    eg = n_edges // _BE
    ng = n_nodes // _BN

    tf = t.astype(_F32)
    f64 = jnp.exp(-_LOG1E4 * jnp.arange(D // 2, dtype=_F32) / (D // 2))
    targ = tf[:, None] * f64
    temb8 = jnp.broadcast_to(
        jnp.concatenate([jnp.sin(targ), jnp.cos(targ)], axis=-1), (8, D))
    tmw0 = jnp.stack([_w(lp["time_mlp"][0]) for lp in layers])
    tmb0 = jnp.stack([_b2d(lp["time_mlp"][0]) for lp in layers])
    tmw1 = jnp.stack([_w(lp["time_mlp"][1]) for lp in layers])
    tmb1 = jnp.stack([_b2d(lp["time_mlp"][1]) for lp in layers])
    trows = pl.pallas_call(
        _tmlp_body,
        grid=(1,),
        in_specs=[_wspec(8, D), _wspec(D, D), _wspec(1, D), _wspec(D, D),
                  _wspec(1, D), _wspec(n_layers, D, D), _wspec(n_layers, 1, D),
                  _wspec(n_layers, D, D), _wspec(n_layers, 1, D)],
        out_specs=_wspec(n_layers, 8, D),
        out_shape=jax.ShapeDtypeStruct((n_layers, 8, D), _F32),
    )(temb8, _w(params["time_proj"][0]), _b2d(params["time_proj"][0]),
      _w(params["time_proj"][1]), _b2d(params["time_proj"][1]),
      tmw0, tmb0, tmw1, tmb1)

    e = pl.pallas_call(
        _embed_e_body,
        grid=(eg,),
        in_specs=[_rowspec(_BE, 1), _rowspec(_BE, 1)],
        out_specs=_rowspec(_BE, D),
        out_shape=jax.ShapeDtypeStruct((n_edges, D), _F32),
    )(edge_distances.reshape(n_edges, 1), x_t.reshape(n_edges, 1))

    n_pad = ((n_nodes + NS * 8 - 1) // (NS * 8)) * (NS * 8)
    zeros_nd = jnp.zeros((n_pad, D), _F32)
    wspec_dd = _wspec(D, D)
    wspec_1d = _wspec(1, D)

    h = uh = None
    agg = None
    for l in range(n_layers):
        lp = layers[l]
        last = l == n_layers - 1
        if l == 0:
            h, a_tab, uh = pl.pallas_call(
                _node0_body,
                grid=(ng,),
                in_specs=[_rowspec(_BN, 2)] + [wspec_dd] * 2,
                out_specs=[_rowspec(_BN, D)] * 3,
                out_shape=[jax.ShapeDtypeStruct((n_nodes, D), _F32)] * 3,
            )(node_coords, _w(lp["Q"]), _w(lp["U"]))
        else:
            nn = layers[l - 1]["node_norm"]
            common = (h, uh, agg[0], agg[1],
                      nn["w"].reshape(1, D), nn["b"].reshape(1, D))
            if last:
                h, a_tab = pl.pallas_call(
                    _node_last_body,
                    grid=(ng,),
                    in_specs=[_rowspec(_BN, D)] * 4 + [wspec_1d] * 2
                             + [wspec_dd],
                    out_specs=[_rowspec(_BN, D)] * 2,
                    out_shape=[jax.ShapeDtypeStruct((n_nodes, D), _F32)] * 2,
                )(*common, _w(lp["Q"]))
            else:
                h, a_tab, uh = pl.pallas_call(
                    _node_body,
                    grid=(ng,),
                    in_specs=[_rowspec(_BN, D)] * 4 + [wspec_1d] * 2
                             + [wspec_dd] * 2,
                    out_specs=[_rowspec(_BN, D)] * 3,
                    out_shape=[jax.ShapeDtypeStruct((n_nodes, D), _F32)] * 3,
                )(*common, _w(lp["Q"]), _w(lp["U"]))

        ga, gb = _gather_tables(a_tab, h, src, dst)

        trow = lax.slice_in_dim(trows[l], 0, 1, axis=0)
        rv = (_w(lp["R"]) if last
              else jnp.concatenate([_w(lp["R"]), _w(lp["V"])], axis=1))
        ew = (_w(lp["P"]), rv, _w(lp["edge_mlp"][0]), _b2d(lp["edge_mlp"][0]),
              _w(lp["edge_mlp"][1]), _b2d(lp["edge_mlp"][1]),
              lp["edge_norm"]["w"].reshape(1, D),
              lp["edge_norm"]["b"].reshape(1, D), trow)
        in_specs = [_rowspec(_BE, D), _rowspec(_BE, D), _rowspec(_BE, D),
                    wspec_dd, _wspec(D, D if last else 2 * D),
                    wspec_dd, wspec_1d, wspec_dd, wspec_1d,
                    wspec_1d, wspec_1d, wspec_1d]
        if last:
            e = pl.pallas_call(
                _edge_last_body,
                grid=(eg,),
                in_specs=in_specs,
                out_specs=_rowspec(_BE, D),
                out_shape=jax.ShapeDtypeStruct((n_edges, D), _F32),
            )(e, ga, gb, *ew)
        else:
            e, msg = pl.pallas_call(
                _edge_body,
                grid=(eg,),
                in_specs=in_specs,
                out_specs=[_rowspec(_BE, D), _rowspec(_BE, D)],
                out_shape=[jax.ShapeDtypeStruct((n_edges, D), _F32),
                           jax.ShapeDtypeStruct((n_edges, D), _F32)],
            )(e, ga, gb, *ew)
            agg = _scatter_msg(msg, src, zeros_nd)

    hw2 = jnp.zeros((D, D), _F32).at[:, :2].set(_w(params["edge_head"][1]))
    hb2 = jnp.zeros((1, D), _F32).at[0, :2].set(params["edge_head"][1]["b"])
    out = pl.pallas_call(
        _head_body,
        grid=(eg,),
        in_specs=[_rowspec(_BE, D), wspec_dd, wspec_1d, wspec_dd, wspec_1d],
        out_specs=_rowspec(_BE, D),
        out_shape=jax.ShapeDtypeStruct((n_edges, D), _F32),
    )(e, _w(params["edge_head"][0]), _b2d(params["edge_head"][0]), hw2, hb2)
    return out[:, :2]
